# baseline (speedup 1.0000x reference)
import jax
import jax.numpy as jnp
import numpy as np
from jax.experimental import pallas as pl
from jax.experimental.pallas import tpu as pltpu

H = 1024
NH = 16
NKV = 4
HD = 64
GROUPS = NH // NKV
MAXPOS = 4096
THETA = 10000.0
I = 512
E = 8
SHARED_I = 512
EPS = 1e-5

BT = 256

_inv_freq = 1.0 / (THETA ** (np.arange(0, HD, 2, dtype=np.float32) / HD))
_t = np.arange(MAXPOS, dtype=np.float32)
_freqs = np.outer(_t, _inv_freq)
_emb = np.concatenate([_freqs, _freqs], axis=-1)
_COS_TAB = jnp.asarray(np.cos(_emb), dtype=jnp.float32)
_SIN_TAB = jnp.asarray(np.sin(_emb), dtype=jnp.float32)


def _rot_half(x):
    h = x.shape[-1] // 2
    return jnp.concatenate([-x[..., h:], x[..., :h]], axis=-1)


def _nrsqrt(v):
    r = jax.lax.rsqrt(v)
    r = r * (1.5 - 0.5 * v * r * r)
    r = r * (1.5 - 0.5 * v * r * r)
    return r


def _qkv_body(x_ref, r_ref, w_ref, ln_ref, cos_ref, sin_ref, q_ref, k_ref, v_ref):
    xb = x_ref[...]
    hb = (xb * r_ref[...]) * ln_ref[...]
    qkv = jnp.dot(hb, w_ref[...], preferred_element_type=jnp.float32)
    q = qkv[:, : NH * HD].reshape(BT, NH, HD)
    k = qkv[:, NH * HD : (NH + NKV) * HD].reshape(BT, NKV, HD)
    v = qkv[:, (NH + NKV) * HD :].reshape(BT, NKV, HD)
    cos = cos_ref[...][:, None, :]
    sin = sin_ref[...][:, None, :]
    q = q * cos + _rot_half(q) * sin
    k = k * cos + _rot_half(k) * sin
    scale = 1.0 / np.sqrt(HD)
    q_ref[...] = (q * scale).transpose(1, 0, 2)
    k_ref[...] = k.transpose(1, 0, 2)
    v_ref[...] = v.transpose(1, 0, 2)


def _qkv_call(x, r1, wqkv_t, ln_w, cos, sin):
    S = x.shape[0]
    return pl.pallas_call(
        _qkv_body,
        grid=(S // BT,),
        in_specs=[
            pl.BlockSpec((BT, H), lambda i: (i, 0)),
            pl.BlockSpec((BT, 1), lambda i: (i, 0)),
            pl.BlockSpec((H, (NH + 2 * NKV) * HD), lambda i: (0, 0)),
            pl.BlockSpec((1, H), lambda i: (0, 0)),
            pl.BlockSpec((BT, HD), lambda i: (i, 0)),
            pl.BlockSpec((BT, HD), lambda i: (i, 0)),
        ],
        out_specs=[
            pl.BlockSpec((NH, BT, HD), lambda i: (0, i, 0)),
            pl.BlockSpec((NKV, BT, HD), lambda i: (0, i, 0)),
            pl.BlockSpec((NKV, BT, HD), lambda i: (0, i, 0)),
        ],
        out_shape=[
            jax.ShapeDtypeStruct((NH, S, HD), jnp.float32),
            jax.ShapeDtypeStruct((NKV, S, HD), jnp.float32),
            jax.ShapeDtypeStruct((NKV, S, HD), jnp.float32),
        ],
    )(x, r1, wqkv_t, ln_w, cos, sin)


def _score_body(q_ref, k_ref, s_ref):
    s_ref[0] = jax.lax.dot_general(q_ref[0], k_ref[0], (((1,), (1,)), ((), ())),
                                   preferred_element_type=jnp.float32)


def _score_call(q, k):
    S = q.shape[1]
    return pl.pallas_call(
        _score_body,
        grid=(NH,),
        in_specs=[
            pl.BlockSpec((1, S, HD), lambda h: (h, 0, 0)),
            pl.BlockSpec((1, S, HD), lambda h: (h // GROUPS, 0, 0)),
        ],
        out_specs=pl.BlockSpec((1, S, S), lambda h: (h, 0, 0)),
        out_shape=jax.ShapeDtypeStruct((NH, S, S), jnp.float32),
    )(q, k)


def _post_body(ao_ref, ow_ref, x0_ref, ln_ref, h1_ref, x_ref):
    ao = ao_ref[...].transpose(1, 0, 2).reshape(BT, NH * HD)
    attn = jnp.dot(ao, ow_ref[...], preferred_element_type=jnp.float32)
    h1 = x0_ref[...] + attn
    h1_ref[...] = h1
    var = jnp.mean(h1 * h1, axis=-1, keepdims=True)
    r = _nrsqrt(var + EPS)
    x_ref[...] = (h1 * r) * ln_ref[...]


def _post_call(ao, ow_t, x0, ln_w):
    S = x0.shape[0]
    return pl.pallas_call(
        _post_body,
        grid=(S // BT,),
        in_specs=[
            pl.BlockSpec((NH, BT, HD), lambda i: (0, i, 0)),
            pl.BlockSpec((NH * HD, H), lambda i: (0, 0)),
            pl.BlockSpec((BT, H), lambda i: (i, 0)),
            pl.BlockSpec((1, H), lambda i: (0, 0)),
        ],
        out_specs=[
            pl.BlockSpec((BT, H), lambda i: (i, 0)),
            pl.BlockSpec((BT, H), lambda i: (i, 0)),
        ],
        out_shape=[
            jax.ShapeDtypeStruct((S, H), jnp.float32),
            jax.ShapeDtypeStruct((S, H), jnp.float32),
        ],
    )(ao, ow_t, x0, ln_w)


NE = E + 1


def _moe_body(x_ref, gup_ref, down_ref, lg_ref, h1_ref, out_ref, acc_ref):
    e = pl.program_id(1)

    @pl.when(e == 0)
    def _():
        acc_ref[...] = h1_ref[...]

    lg = lg_ref[...]
    idx = jax.lax.broadcasted_iota(jnp.int32, (BT, E), 1)
    m1 = jnp.max(lg, axis=-1, keepdims=True)
    i1 = jnp.min(jnp.where(lg == m1, idx, E), axis=-1, keepdims=True)
    masked = jnp.where(idx == i1, -jnp.inf, lg)
    m2 = jnp.max(masked, axis=-1, keepdims=True)
    i2 = jnp.min(jnp.where(masked == m2, idx, E), axis=-1, keepdims=True)
    s1 = jax.nn.sigmoid(m1)
    s2 = jax.nn.sigmoid(m2)
    denom = s1 + s2 + 1e-20
    we = (jnp.where(i1 == e, s1 / denom, 0.0)
          + jnp.where(i2 == e, s2 / denom, 0.0)
          + jnp.where(e == E, 1.0, 0.0))

    xb = x_ref[...]
    gup = gup_ref[0]
    g = jnp.dot(xb, gup[:, :I], preferred_element_type=jnp.float32)
    u = jnp.dot(xb, gup[:, I:], preferred_element_type=jnp.float32)
    act = (g * jax.nn.sigmoid(g)) * u
    y = jnp.dot(act, down_ref[0], preferred_element_type=jnp.float32)
    acc_ref[...] += we * y

    @pl.when(e == NE - 1)
    def _():
        out_ref[...] = acc_ref[...]


def _moe_call(x, gup_t, down_t, logits, h1):
    S = x.shape[0]
    return pl.pallas_call(
        _moe_body,
        grid=(S // BT, NE),
        in_specs=[
            pl.BlockSpec((BT, H), lambda t, e: (t, 0)),
            pl.BlockSpec((1, H, 2 * I), lambda t, e: (e, 0, 0)),
            pl.BlockSpec((1, I, H), lambda t, e: (e, 0, 0)),
            pl.BlockSpec((BT, E), lambda t, e: (t, 0)),
            pl.BlockSpec((BT, H), lambda t, e: (t, 0)),
        ],
        out_specs=pl.BlockSpec((BT, H), lambda t, e: (t, 0)),
        out_shape=jax.ShapeDtypeStruct((S, H), jnp.float32),
        scratch_shapes=[pltpu.VMEM((BT, H), jnp.float32)],
    )(x, gup_t, down_t, logits, h1)


def kernel(hidden_states, position_ids, input_ln_w, post_ln_w, q_w, k_w, v_w,
           o_w, gate_weight, experts_gate_up, experts_down, shared_gate_w,
           shared_up_w, shared_down_w):
    Bx, S, _ = hidden_states.shape
    x0 = hidden_states.reshape(S, H)
    cos = _COS_TAB[position_ids[0]]
    sin = _SIN_TAB[position_ids[0]]

    var1 = jnp.mean(x0 * x0, axis=-1, keepdims=True)
    r1 = jax.lax.rsqrt(var1 + EPS)

    wqkv_t = jnp.concatenate([q_w, k_w, v_w], axis=0).T
    q, k, v = _qkv_call(x0, r1, wqkv_t, input_ln_w.reshape(1, H), cos, sin)
    s = _score_call(q, k)[None]
    mask = jnp.triu(jnp.full((S, S), -jnp.inf, dtype=jnp.float32), 1)
    aw = jax.nn.softmax(s + mask[None, None, :, :], axis=-1)
    vv = jnp.repeat(v[None], GROUPS, axis=1)
    ao = jnp.einsum('bhqk,bhkd->bhqd', aw, vv)[0]
    h1, x = _post_call(ao, o_w.T, x0, post_ln_w.reshape(1, H))
    logits = x @ gate_weight.T

    shared_gup = jnp.concatenate([shared_gate_w, shared_up_w], axis=0)
    gup_all = jnp.concatenate(
        [experts_gate_up, shared_gup[None]], axis=0
    ).transpose(0, 2, 1)
    down_all = jnp.concatenate(
        [experts_down, shared_down_w[None]], axis=0
    ).transpose(0, 2, 1)

    out = _moe_call(x, gup_all, down_all, logits, h1)
    return out.reshape(Bx, S, H)
